# baseline (speedup 1.0000x reference)
import jax
import jax.numpy as jnp
from jax.experimental import pallas as pl
from jax.experimental.pallas import tpu as pltpu

ALPHA_ = 0.1
BM = 512


def _prop_kernel(adj_ref, x_ref, h_ref, o_ref):
    i = pl.program_id(0)
    o_ref[...] = (1.0 - ALPHA_) * jnp.dot(
        adj_ref[...], x_ref[...], preferred_element_type=jnp.float32
    ) + ALPHA_ * h_ref[pl.ds((i % 4) * BM, BM), :]


@jax.jit
def kernel(x, adj, h):
    n, d = x.shape
    nm = n // BM
    return pl.pallas_call(
        _prop_kernel,
        grid=(nm,),
        in_specs=[
            pl.BlockSpec((BM, n), lambda i: (i, 0)),
            pl.BlockSpec((n, d), lambda i: (0, 0)),
            pl.BlockSpec((n // 2, d), lambda i: (i // 4, 0)),
        ],
        out_specs=pl.BlockSpec((BM, d), lambda i: (i, 0)),
        out_shape=jax.ShapeDtypeStruct((n, d), jnp.float32),
        compiler_params=pltpu.CompilerParams(
            dimension_semantics=("arbitrary",),
        ),
    )(adj, x, h)


# speedup vs baseline: 1.1025x; 1.1025x over previous
import jax
import jax.numpy as jnp
from jax.experimental import pallas as pl
from jax.experimental.pallas import tpu as pltpu

ALPHA_ = 0.1
BM = 512


def _prop_kernel(adj_ref, x_ref, h_ref, o_ref):
    i = pl.program_id(0)
    o_ref[...] = (1.0 - ALPHA_) * jnp.dot(
        adj_ref[...], x_ref[...], preferred_element_type=jnp.float32
    ) + ALPHA_ * h_ref[pl.ds(i * BM, BM), :]


@jax.jit
def kernel(x, adj, h):
    n, d = x.shape
    nm = n // BM
    return pl.pallas_call(
        _prop_kernel,
        grid=(nm,),
        in_specs=[
            pl.BlockSpec((BM, n), lambda i: (i, 0)),
            pl.BlockSpec((n, d), lambda i: (0, 0)),
            pl.BlockSpec((n, d), lambda i: (0, 0)),
        ],
        out_specs=pl.BlockSpec((BM, d), lambda i: (i, 0)),
        out_shape=jax.ShapeDtypeStruct((n, d), jnp.float32),
        compiler_params=pltpu.CompilerParams(
            dimension_semantics=("arbitrary",),
        ),
    )(adj, x, h)
